# baseline (speedup 1.0000x reference)
import functools

import jax
import jax.numpy as jnp
from jax import lax
from jax.experimental import pallas as pl
from jax.experimental.pallas import tpu as pltpu
from jax.experimental.pallas import tpu_sc as plsc

N = 10000
NP = 10240
BLK = 1024
GRID = NP // BLK
F_IN = 128
H = 256
SW = 128
NSLAB = 2
KH = 512
G = 64
C = 10
E = 320000
NSUB = 16
CHUNK = 128
CPT = 160
GSZ = 2
NGRP = CPT // GSZ
NITER = NGRP // 2
EPT = CPT * CHUNK
EPAD = EPT * NSUB
DUMMY = N
RPT = NP // NSUB


def _split_out(c, out_ref):
    for k in range(NSLAB):
        out_ref[k] = c[:, k * SW:(k + 1) * SW].astype(jnp.bfloat16)


def _first_body(x_ref, w0_ref, b0_ref, w1_ref, b1_ref, w2_ref, out_ref):
    h = jnp.dot(x_ref[...], w0_ref[...], preferred_element_type=jnp.float32)
    h = h + b0_ref[...]
    m = jnp.dot(h, w1_ref[...], preferred_element_type=jnp.float32) + b1_ref[...]
    m = jnp.maximum(m, 0.0)
    c = jnp.dot(m, w2_ref[...], preferred_element_type=jnp.float32)
    _split_out(c, out_ref)


def _mid_body(s_ref, b2p_ref, w1_ref, b1_ref, w2_ref, out_ref):
    hcat = jnp.concatenate([s_ref[k] for k in range(NSLAB)],
                           axis=-1).astype(jnp.float32)
    h = jnp.maximum(hcat + b2p_ref[...], 0.0)
    m = jnp.dot(h, w1_ref[...], preferred_element_type=jnp.float32) + b1_ref[...]
    m = jnp.maximum(m, 0.0)
    c = jnp.dot(m, w2_ref[...], preferred_element_type=jnp.float32)
    _split_out(c, out_ref)


def _final_body(s_ref, b2p_ref, batch_ref, w1_ref, b1_ref, w2_ref, b2f_ref,
                out_ref, pooled_acc, cnt_acc):
    i = pl.program_id(0)

    @pl.when(i == 0)
    def _():
        pooled_acc[...] = jnp.zeros_like(pooled_acc)
        cnt_acc[...] = jnp.zeros_like(cnt_acc)

    hcat = jnp.concatenate([s_ref[k] for k in range(NSLAB)],
                           axis=-1).astype(jnp.float32)
    h = jnp.maximum(hcat + b2p_ref[...], 0.0)
    b = batch_ref[0]
    bb = jnp.broadcast_to(b, (G, BLK))
    gg = lax.broadcasted_iota(jnp.int32, (G, BLK), 0)
    oh = (gg == bb).astype(jnp.float32)
    pooled_acc[...] += jnp.dot(oh, h, preferred_element_type=jnp.float32)
    cnt_acc[...] += jnp.dot(oh, jnp.ones((BLK, 128), jnp.float32),
                            preferred_element_type=jnp.float32)

    @pl.when(i == GRID - 1)
    def _():
        cnt = jnp.maximum(cnt_acc[:, :1], 1.0)
        pooled = pooled_acc[...] / cnt
        z = jnp.dot(pooled, w1_ref[...], preferred_element_type=jnp.float32)
        z = jnp.maximum(z + b1_ref[...], 0.0)
        logits = jnp.dot(z, w2_ref[...], preferred_element_type=jnp.float32)
        logits = logits + b2f_ref[...]
        mx = jnp.max(logits, axis=-1, keepdims=True)
        lse = jnp.log(jnp.sum(jnp.exp(logits - mx), axis=-1, keepdims=True)) + mx
        out_ref[...] = logits - lse


def _full(shape):
    return pl.BlockSpec(shape, lambda i: tuple(0 for _ in shape))


_first_call = pl.pallas_call(
    _first_body,
    grid=(GRID,),
    in_specs=[
        pl.BlockSpec((BLK, F_IN), lambda i: (i, 0)),
        _full((F_IN, H)),
        _full((1, H)),
        _full((H, KH)),
        _full((1, KH)),
        _full((KH, H)),
    ],
    out_specs=pl.BlockSpec((NSLAB, BLK, SW), lambda i: (0, i, 0)),
    out_shape=jax.ShapeDtypeStruct((NSLAB, NP, SW), jnp.bfloat16),
)

_mid_call = pl.pallas_call(
    _mid_body,
    grid=(GRID,),
    in_specs=[
        pl.BlockSpec((NSLAB, BLK, SW), lambda i: (0, i, 0)),
        _full((1, H)),
        _full((H, KH)),
        _full((1, KH)),
        _full((KH, H)),
    ],
    out_specs=pl.BlockSpec((NSLAB, BLK, SW), lambda i: (0, i, 0)),
    out_shape=jax.ShapeDtypeStruct((NSLAB, NP, SW), jnp.bfloat16),
)

_final_call = pl.pallas_call(
    _final_body,
    grid=(GRID,),
    in_specs=[
        pl.BlockSpec((NSLAB, BLK, SW), lambda i: (0, i, 0)),
        _full((1, H)),
        pl.BlockSpec((1, 1, BLK), lambda i: (i, 0, 0)),
        _full((H, H)),
        _full((1, H)),
        _full((H, C)),
        _full((1, C)),
    ],
    out_specs=_full((G, C)),
    out_shape=jax.ShapeDtypeStruct((G, C), jnp.float32),
    scratch_shapes=[
        pltpu.VMEM((G, H), jnp.float32),
        pltpu.VMEM((G, 128), jnp.float32),
    ],
)


def _agg_body(c_hbm, src_hbm, dst_hbm, out_hbm, src_v, dst_v, rows0, rows1,
              acc_sh, sem_a, sem_b):
    cid = lax.axis_index("c")
    sid = lax.axis_index("s")

    pltpu.sync_copy(src_hbm.at[sid], src_v)
    pltpu.sync_copy(dst_hbm.at[sid], dst_v)
    pltpu.sync_copy(c_hbm.at[cid, pl.ds(sid * RPT, RPT)],
                    acc_sh.at[pl.ds(sid * RPT, RPT)])
    plsc.subcore_barrier()

    def gather(g, buf, sem):
        return [pltpu.make_async_copy(c_hbm.at[cid].at[src_v.at[g * GSZ + t]],
                                      buf.at[pl.ds(t * CHUNK, CHUNK)], sem)
                for t in range(GSZ)]

    def gather_start(g, buf, sem):
        for d in gather(g, buf, sem):
            d.start()

    def gather_wait(g, buf, sem):
        for d in gather(g, buf, sem):
            d.wait()

    def scat(g, buf):
        for t in range(GSZ):
            pltpu.sync_copy(buf.at[pl.ds(t * CHUNK, CHUNK)],
                            acc_sh.at[dst_v.at[g * GSZ + t]], add=True)

    gather_start(0, rows0, sem_a)

    def body(k, carry):
        g0 = 2 * k
        g1 = g0 + 1
        gather_start(g1, rows1, sem_b)
        gather_wait(g0, rows0, sem_a)
        scat(g0, rows0)
        g2 = jnp.where(k < NITER - 1, g0 + 2, 0)
        gather_start(g2, rows0, sem_a)
        gather_wait(g1, rows1, sem_b)
        scat(g1, rows1)
        return carry

    lax.fori_loop(0, NITER, body, 0, unroll=False)
    gather_wait(0, rows0, sem_a)
    plsc.subcore_barrier()
    pltpu.sync_copy(acc_sh.at[pl.ds(sid * RPT, RPT)],
                    out_hbm.at[cid, pl.ds(sid * RPT, RPT)])


@functools.cache
def _get_agg_call():
    return functools.partial(
        pl.kernel,
        out_type=jax.ShapeDtypeStruct((NSLAB, NP, SW), jnp.bfloat16),
        mesh=plsc.VectorSubcoreMesh(core_axis_name="c", subcore_axis_name="s"),
        compiler_params=pltpu.CompilerParams(use_tc_tiling_on_sc=False),
        scratch_types=[
            pltpu.VMEM((CPT, CHUNK), jnp.int32),
            pltpu.VMEM((CPT, CHUNK), jnp.int32),
            pltpu.VMEM((GSZ * CHUNK, SW), jnp.bfloat16),
            pltpu.VMEM((GSZ * CHUNK, SW), jnp.bfloat16),
            pltpu.VMEM_SHARED((NP, SW), jnp.bfloat16),
            pltpu.SemaphoreType.DMA,
            pltpu.SemaphoreType.DMA,
        ],
    )(_agg_body)


def kernel(x, edge_index, batch, lin0_W, lin0_b, conv_W1, conv_b1, conv_W2,
           conv_b2, lin1_W, lin1_b, lin2_W, lin2_b):
    xp = jnp.concatenate([x, jnp.zeros((NP - N, F_IN), jnp.float32)], axis=0)
    src = jnp.concatenate(
        [edge_index[0], jnp.zeros((EPAD - E,), jnp.int32)]).reshape(NSUB, CPT, CHUNK)
    dst = jnp.concatenate(
        [edge_index[1], jnp.full((EPAD - E,), DUMMY, jnp.int32)]).reshape(NSUB, CPT, CHUNK)
    batchp = jnp.concatenate(
        [batch, jnp.full((NP - N,), G, jnp.int32)]).reshape(GRID, 1, BLK)

    agg = _get_agg_call()
    b0 = lin0_b.reshape(1, H)
    c = _first_call(xp, lin0_W, b0, conv_W1[0], conv_b1[0].reshape(1, KH),
                    conv_W2[0])
    s = agg(c, src, dst)
    c = _mid_call(s, conv_b2[0].reshape(1, H), conv_W1[1],
                  conv_b1[1].reshape(1, KH), conv_W2[1])
    s = agg(c, src, dst)
    c = _mid_call(s, conv_b2[1].reshape(1, H), conv_W1[2],
                  conv_b1[2].reshape(1, KH), conv_W2[2])
    s = agg(c, src, dst)
    out = _final_call(s, conv_b2[2].reshape(1, H), batchp, lin1_W,
                      lin1_b.reshape(1, H), lin2_W, lin2_b.reshape(1, C))
    return out


# speedup vs baseline: 1.3506x; 1.3506x over previous
import functools

import jax
import jax.numpy as jnp
from jax import lax
from jax.experimental import pallas as pl
from jax.experimental.pallas import tpu as pltpu
from jax.experimental.pallas import tpu_sc as plsc

N = 10000
NP = 10240
BLK = 2048
GRID = NP // BLK
F_IN = 128
H = 256
SW = 128
NSLAB = 2
KH = 512
G = 64
C = 10
E = 320000
NSUB = 16
CHUNK = 128
CPT = 158
NHALF = CPT // 2
EPT = CPT * CHUNK
EPAD = EPT * NSUB
DUMMY = N
RPT = NP // NSUB


def _split_out(c, out_ref):
    for k in range(NSLAB):
        out_ref[k] = c[:, k * SW:(k + 1) * SW].astype(jnp.bfloat16)


def _first_body(x_ref, w0_ref, b0_ref, w1_ref, b1_ref, w2_ref, out_ref):
    h = jnp.dot(x_ref[...], w0_ref[...], preferred_element_type=jnp.float32)
    h = h + b0_ref[...]
    m = jnp.dot(h, w1_ref[...], preferred_element_type=jnp.float32) + b1_ref[...]
    m = jnp.maximum(m, 0.0)
    c = jnp.dot(m, w2_ref[...], preferred_element_type=jnp.float32)
    _split_out(c, out_ref)


def _mid_body(s_ref, b2p_ref, w1_ref, b1_ref, w2_ref, out_ref):
    hcat = jnp.concatenate([s_ref[k] for k in range(NSLAB)],
                           axis=-1).astype(jnp.float32)
    h = jnp.maximum(hcat + b2p_ref[...], 0.0)
    m = jnp.dot(h, w1_ref[...], preferred_element_type=jnp.float32) + b1_ref[...]
    m = jnp.maximum(m, 0.0)
    c = jnp.dot(m, w2_ref[...], preferred_element_type=jnp.float32)
    _split_out(c, out_ref)


def _final_body(s_ref, b2p_ref, batch_ref, w1_ref, b1_ref, w2_ref, b2f_ref,
                out_ref, pooled_acc, cnt_acc):
    i = pl.program_id(0)

    @pl.when(i == 0)
    def _():
        pooled_acc[...] = jnp.zeros_like(pooled_acc)
        cnt_acc[...] = jnp.zeros_like(cnt_acc)

    hcat = jnp.concatenate([s_ref[k] for k in range(NSLAB)],
                           axis=-1).astype(jnp.float32)
    h = jnp.maximum(hcat + b2p_ref[...], 0.0)
    b = batch_ref[0]
    bb = jnp.broadcast_to(b, (G, BLK))
    gg = lax.broadcasted_iota(jnp.int32, (G, BLK), 0)
    oh = (gg == bb).astype(jnp.float32)
    pooled_acc[...] += jnp.dot(oh, h, preferred_element_type=jnp.float32)
    cnt_acc[...] += jnp.dot(oh, jnp.ones((BLK, 128), jnp.float32),
                            preferred_element_type=jnp.float32)

    @pl.when(i == GRID - 1)
    def _():
        cnt = jnp.maximum(cnt_acc[:, :1], 1.0)
        pooled = pooled_acc[...] / cnt
        z = jnp.dot(pooled, w1_ref[...], preferred_element_type=jnp.float32)
        z = jnp.maximum(z + b1_ref[...], 0.0)
        logits = jnp.dot(z, w2_ref[...], preferred_element_type=jnp.float32)
        logits = logits + b2f_ref[...]
        mx = jnp.max(logits, axis=-1, keepdims=True)
        lse = jnp.log(jnp.sum(jnp.exp(logits - mx), axis=-1, keepdims=True)) + mx
        out_ref[...] = logits - lse


def _full(shape):
    return pl.BlockSpec(shape, lambda i: tuple(0 for _ in shape))


_first_call = pl.pallas_call(
    _first_body,
    grid=(GRID,),
    in_specs=[
        pl.BlockSpec((BLK, F_IN), lambda i: (i, 0)),
        _full((F_IN, H)),
        _full((1, H)),
        _full((H, KH)),
        _full((1, KH)),
        _full((KH, H)),
    ],
    out_specs=pl.BlockSpec((NSLAB, BLK, SW), lambda i: (0, i, 0)),
    out_shape=jax.ShapeDtypeStruct((NSLAB, NP, SW), jnp.bfloat16),
)

_mid_call = pl.pallas_call(
    _mid_body,
    grid=(GRID,),
    in_specs=[
        pl.BlockSpec((NSLAB, BLK, SW), lambda i: (0, i, 0)),
        _full((1, H)),
        _full((H, KH)),
        _full((1, KH)),
        _full((KH, H)),
    ],
    out_specs=pl.BlockSpec((NSLAB, BLK, SW), lambda i: (0, i, 0)),
    out_shape=jax.ShapeDtypeStruct((NSLAB, NP, SW), jnp.bfloat16),
)

_final_call = pl.pallas_call(
    _final_body,
    grid=(GRID,),
    in_specs=[
        pl.BlockSpec((NSLAB, BLK, SW), lambda i: (0, i, 0)),
        _full((1, H)),
        pl.BlockSpec((1, 1, BLK), lambda i: (i, 0, 0)),
        _full((H, H)),
        _full((1, H)),
        _full((H, C)),
        _full((1, C)),
    ],
    out_specs=_full((G, C)),
    out_shape=jax.ShapeDtypeStruct((G, C), jnp.float32),
    scratch_shapes=[
        pltpu.VMEM((G, H), jnp.float32),
        pltpu.VMEM((G, 128), jnp.float32),
    ],
)


def _agg_body(c_hbm, src_hbm, dst_hbm, out_hbm, src_v, dst_v, rows0, rows1,
              acc_sh, sem_a, sem_b):
    cid = lax.axis_index("c")
    sid = lax.axis_index("s")

    pltpu.sync_copy(src_hbm.at[sid], src_v)
    pltpu.sync_copy(dst_hbm.at[sid], dst_v)
    pltpu.sync_copy(c_hbm.at[cid, pl.ds(sid * RPT, RPT)],
                    acc_sh.at[pl.ds(sid * RPT, RPT)])
    plsc.subcore_barrier()

    def gather(j, buf, sem):
        return pltpu.make_async_copy(c_hbm.at[cid].at[src_v.at[j]], buf, sem)

    gather(0, rows0, sem_a).start()

    def body(k, carry):
        j0 = 2 * k
        j1 = j0 + 1
        gather(j1, rows1, sem_b).start()
        gather(j0, rows0, sem_a).wait()
        pltpu.sync_copy(rows0, acc_sh.at[dst_v.at[j0]], add=True)
        j2 = jnp.where(k < NHALF - 1, j0 + 2, 0)
        gather(j2, rows0, sem_a).start()
        gather(j1, rows1, sem_b).wait()
        pltpu.sync_copy(rows1, acc_sh.at[dst_v.at[j1]], add=True)
        return carry

    lax.fori_loop(0, NHALF, body, 0, unroll=False)
    gather(0, rows0, sem_a).wait()
    plsc.subcore_barrier()
    pltpu.sync_copy(acc_sh.at[pl.ds(sid * RPT, RPT)],
                    out_hbm.at[cid, pl.ds(sid * RPT, RPT)])


@functools.cache
def _get_agg_call():
    return functools.partial(
        pl.kernel,
        out_type=jax.ShapeDtypeStruct((NSLAB, NP, SW), jnp.bfloat16),
        mesh=plsc.VectorSubcoreMesh(core_axis_name="c", subcore_axis_name="s"),
        compiler_params=pltpu.CompilerParams(use_tc_tiling_on_sc=False),
        scratch_types=[
            pltpu.VMEM((CPT, CHUNK), jnp.int32),
            pltpu.VMEM((CPT, CHUNK), jnp.int32),
            pltpu.VMEM((CHUNK, SW), jnp.bfloat16),
            pltpu.VMEM((CHUNK, SW), jnp.bfloat16),
            pltpu.VMEM_SHARED((NP, SW), jnp.bfloat16),
            pltpu.SemaphoreType.DMA,
            pltpu.SemaphoreType.DMA,
        ],
    )(_agg_body)


def kernel(x, edge_index, batch, lin0_W, lin0_b, conv_W1, conv_b1, conv_W2,
           conv_b2, lin1_W, lin1_b, lin2_W, lin2_b):
    xp = jnp.concatenate([x, jnp.zeros((NP - N, F_IN), jnp.float32)], axis=0)
    src = jnp.concatenate(
        [edge_index[0], jnp.zeros((EPAD - E,), jnp.int32)]).reshape(NSUB, CPT, CHUNK)
    dst = jnp.concatenate(
        [edge_index[1], jnp.full((EPAD - E,), DUMMY, jnp.int32)]).reshape(NSUB, CPT, CHUNK)
    batchp = jnp.concatenate(
        [batch, jnp.full((NP - N,), G, jnp.int32)]).reshape(GRID, 1, BLK)

    agg = _get_agg_call()
    b0 = lin0_b.reshape(1, H)
    c = _first_call(xp, lin0_W, b0, conv_W1[0], conv_b1[0].reshape(1, KH),
                    conv_W2[0])
    s = agg(c, src, dst)
    c = _mid_call(s, conv_b2[0].reshape(1, H), conv_W1[1],
                  conv_b1[1].reshape(1, KH), conv_W2[1])
    s = agg(c, src, dst)
    c = _mid_call(s, conv_b2[1].reshape(1, H), conv_W1[2],
                  conv_b1[2].reshape(1, KH), conv_W2[2])
    s = agg(c, src, dst)
    out = _final_call(s, conv_b2[2].reshape(1, H), batchp, lin1_W,
                      lin1_b.reshape(1, H), lin2_W, lin2_b.reshape(1, C))
    return out
